# baseline (speedup 1.0000x reference)
import functools

import jax
import jax.numpy as jnp
from jax import lax
from jax.experimental import pallas as pl
from jax.experimental.pallas import tpu as pltpu
from jax.experimental.pallas import tpu_sc as plsc

_N = 10000
_NP = 10240
_E = 160000
_NG = 64
_NPRED = 10
_H = 512
_LANES = 128
_TILES = 16
_CORES = 2
_EB = 128
_NB = 80
_GB = 16
_NG_IDX = _NB // _GB
_EPT = _NB * _EB
_EPAD = _EPT * _TILES
_RPT = _NP // _TILES


def _make_segsum(C):
    mesh = plsc.VectorSubcoreMesh(core_axis_name="c", subcore_axis_name="s")

    @functools.partial(
        pl.kernel,
        mesh=mesh,
        out_type=jax.ShapeDtypeStruct((C, _NP, _LANES), jnp.float32),
        scratch_types=[
            pltpu.VMEM((2, _GB, _EB), jnp.int32),
            pltpu.VMEM((2, _GB, _EB), jnp.int32),
            pltpu.VMEM((2, _EB, _LANES), jnp.float32),
            pltpu.VMEM((32, _LANES), jnp.float32),
            pltpu.VMEM_SHARED((_NP, _LANES), jnp.float32),
            pltpu.SemaphoreType.DMA,
            pltpu.SemaphoreType.DMA,
            pltpu.SemaphoreType.DMA,
            pltpu.SemaphoreType.DMA,
        ],
    )
    def segsum(h_hbm, src_hbm, dst_hbm, out_hbm,
               src_v, dst_v, rows_v, zbuf, acc, sem0, sem1, semi0, semi1):
        sems = (sem0, sem1)
        semi = (semi0, semi1)
        cid = lax.axis_index("c")
        sid = lax.axis_index("s")

        def _stage(g, slot):
            pltpu.make_async_copy(src_hbm.at[sid].at[g],
                                  src_v.at[slot], semi[slot]).start()
            pltpu.make_async_copy(dst_hbm.at[sid].at[g],
                                  dst_v.at[slot], semi[slot]).start()

        def _stage_wait(g, slot):
            pltpu.make_async_copy(src_hbm.at[sid].at[g],
                                  src_v.at[slot], semi[slot]).wait()
            pltpu.make_async_copy(dst_hbm.at[sid].at[g],
                                  dst_v.at[slot], semi[slot]).wait()

        zv = jnp.zeros((16,), jnp.float32)

        def _zb(i, carry):
            r = i // (_LANES // 16)
            col = (i % (_LANES // 16)) * 16
            zbuf[r, pl.ds(col, 16)] = zv
            return carry

        lax.fori_loop(0, 32 * (_LANES // 16), _zb, 0)

        def _edge_loop(c):
            tbl = h_hbm.at[c]

            def _start(j, slot, buf):
                for h in range(2):
                    half = pl.ds(64 * h, 64)
                    pltpu.make_async_copy(
                        tbl.at[src_v.at[slot].at[j].at[half]],
                        rows_v.at[buf].at[half], sems[buf]).start()

            def _finish(j, slot, buf):
                for h in range(2):
                    half = pl.ds(64 * h, 64)
                    pltpu.make_async_copy(
                        tbl.at[src_v.at[slot].at[j].at[half]],
                        rows_v.at[buf].at[half], sems[buf]).wait()
                pltpu.sync_copy(rows_v.at[buf], acc.at[dst_v.at[slot].at[j]],
                                add=True)

            _stage(0, 0)
            for g in range(_NG_IDX):
                slot = g % 2
                _stage_wait(g, slot)
                if g + 1 < _NG_IDX:
                    _stage(g + 1, (g + 1) % 2)
                _start(0, slot, 0)

                def _pair(p, carry, slot=slot):
                    j0 = 2 * p
                    _start(j0 + 1, slot, 1)
                    _finish(j0, slot, 0)

                    @pl.when(j0 + 2 < _GB)
                    def _():
                        _start(j0 + 2, slot, 0)

                    _finish(j0 + 1, slot, 1)
                    return carry

                lax.fori_loop(0, _GB // 2, _pair, 0)

        def _writeout(c):
            base = sid * _RPT
            for j in range(_RPT // _EB):
                pltpu.sync_copy(acc.at[pl.ds(base + j * _EB, _EB)],
                                out_hbm.at[c].at[pl.ds(base + j * _EB, _EB)])

        for k in range(C // _CORES):
            base = sid * _RPT
            for j in range(_RPT // 32):
                pltpu.sync_copy(zbuf, acc.at[pl.ds(base + j * 32, 32)])
            plsc.subcore_barrier()

            for c in (2 * k, 2 * k + 1):
                pl.when(cid == (c % 2))(functools.partial(_edge_loop, c))
            plsc.subcore_barrier()

            for c in (2 * k, 2 * k + 1):
                pl.when(cid == (c % 2))(functools.partial(_writeout, c))
            if k + 1 < C // _CORES:
                plsc.subcore_barrier()

    return segsum


def _make_dense(C_in, relu, BM=1024):
    C_out = _H // _LANES

    def body(agg_ref, h_ref, wrel_ref, wroot_ref, b_ref, out_ref):
        s = jnp.zeros((BM, _H), jnp.float32)
        for c in range(C_in):
            s = s + jnp.dot(agg_ref[c], wrel_ref[c],
                            preferred_element_type=jnp.float32)
            s = s + jnp.dot(h_ref[c], wroot_ref[c],
                            preferred_element_type=jnp.float32)
        s = s + b_ref[...]
        if relu:
            s = jnp.maximum(s, 0.0)
        for co in range(C_out):
            out_ref[co] = s[:, co * _LANES:(co + 1) * _LANES]

    return pl.pallas_call(
        body,
        grid=(_NP // BM,),
        in_specs=[
            pl.BlockSpec((C_in, BM, _LANES), lambda i: (0, i, 0)),
            pl.BlockSpec((C_in, BM, _LANES), lambda i: (0, i, 0)),
            pl.BlockSpec((C_in, _LANES, _H), lambda i: (0, 0, 0)),
            pl.BlockSpec((C_in, _LANES, _H), lambda i: (0, 0, 0)),
            pl.BlockSpec((1, _H), lambda i: (0, 0)),
        ],
        out_specs=pl.BlockSpec((C_out, BM, _LANES), lambda i: (0, i, 0)),
        out_shape=jax.ShapeDtypeStruct((C_out, _NP, _LANES), jnp.float32),
    )


def _make_chunk2(BM=1024):
    def body(x_ref, out_ref):
        out_ref[0] = x_ref[...]

    return pl.pallas_call(
        body,
        grid=(2, _NP // BM),
        in_specs=[pl.BlockSpec((BM, _LANES), lambda c, i: (i, c))],
        out_specs=pl.BlockSpec((1, BM, _LANES), lambda c, i: (c, i, 0)),
        out_shape=jax.ShapeDtypeStruct((2, _NP, _LANES), jnp.float32),
    )


def _make_pool(BM=1024):
    nsteps = _NP // BM
    C_in = _H // _LANES

    def body(agg_ref, h_ref, wrel_ref, wroot_ref, brel_ref, batch_ref,
             wlin_ref, blin_ref, out_ref, sums_ref):
        i = pl.program_id(0)

        @pl.when(i == 0)
        def _():
            sums_ref[...] = jnp.zeros_like(sums_ref)

        s = jnp.zeros((BM, _H), jnp.float32)
        for c in range(C_in):
            s = s + jnp.dot(agg_ref[c], wrel_ref[c],
                            preferred_element_type=jnp.float32)
            s = s + jnp.dot(h_ref[c], wroot_ref[c],
                            preferred_element_type=jnp.float32)
        s = s + brel_ref[...]

        oh = (batch_ref[...] ==
              lax.broadcasted_iota(jnp.int32, (1, _NG), 1)).astype(jnp.float32)
        hcat = jnp.concatenate([s, jnp.ones((BM, _LANES), jnp.float32)],
                               axis=1)
        sums_ref[...] += lax.dot_general(
            oh, hcat, (((0,), (0,)), ((), ())),
            preferred_element_type=jnp.float32)

        @pl.when(i == nsteps - 1)
        def _():
            t = sums_ref[...]
            pooled = t[:, :_H] / jnp.maximum(t[:, _H:_H + 1], 1.0)
            out_ref[...] = (jnp.dot(pooled, wlin_ref[...],
                                    preferred_element_type=jnp.float32)
                            + blin_ref[...])

    return pl.pallas_call(
        body,
        grid=(nsteps,),
        in_specs=[
            pl.BlockSpec((C_in, BM, _LANES), lambda i: (0, i, 0)),
            pl.BlockSpec((C_in, BM, _LANES), lambda i: (0, i, 0)),
            pl.BlockSpec((C_in, _LANES, _H), lambda i: (0, 0, 0)),
            pl.BlockSpec((C_in, _LANES, _H), lambda i: (0, 0, 0)),
            pl.BlockSpec((1, _H), lambda i: (0, 0)),
            pl.BlockSpec((BM, 1), lambda i: (i, 0)),
            pl.BlockSpec((_H, _NPRED), lambda i: (0, 0)),
            pl.BlockSpec((1, _NPRED), lambda i: (0, 0)),
        ],
        out_specs=pl.BlockSpec((_NG, _NPRED), lambda i: (0, 0)),
        out_shape=jax.ShapeDtypeStruct((_NG, _NPRED), jnp.float32),
        scratch_shapes=[pltpu.VMEM((_NG, _H + _LANES), jnp.float32)],
    )


def kernel(x, edge_index, batch,
           Wrel1, brel1, Wroot1,
           Wrel2, brel2, Wroot2,
           Wrel3, brel3, Wroot3,
           Wlin, blin):
    src = edge_index[0].astype(jnp.int32)
    dst = edge_index[1].astype(jnp.int32)

    pad = _EPAD - _E
    ar = jnp.arange(pad, dtype=jnp.int32)
    src16 = jnp.concatenate([src, (ar * 997) % _N]).reshape(
        _TILES, _NG_IDX, _GB, _EB)
    dst16 = jnp.concatenate([dst, _N + (ar % 16)]).reshape(
        _TILES, _NG_IDX, _GB, _EB)

    x_pad = jnp.concatenate(
        [x, jnp.zeros((_NP - _N, x.shape[1]), jnp.float32)])
    x_c = _make_chunk2()(x_pad)
    batch_p = jnp.concatenate(
        [batch.astype(jnp.int32), jnp.full((_NP - _N,), _NG, jnp.int32)]
    ).reshape(_NP, 1)

    segsum2 = _make_segsum(2)
    segsum4 = _make_segsum(4)

    agg1 = segsum2(x_c, src16, dst16)
    h1 = _make_dense(2, True)(agg1, x_c,
                              Wrel1.reshape(2, _LANES, _H),
                              Wroot1.reshape(2, _LANES, _H),
                              brel1.reshape(1, _H))
    agg2 = segsum4(h1, src16, dst16)
    h2 = _make_dense(4, True)(agg2, h1,
                              Wrel2.reshape(4, _LANES, _H),
                              Wroot2.reshape(4, _LANES, _H),
                              brel2.reshape(1, _H))
    agg3 = segsum4(h2, src16, dst16)
    out = _make_pool()(agg3, h2,
                       Wrel3.reshape(4, _LANES, _H),
                       Wroot3.reshape(4, _LANES, _H),
                       brel3.reshape(1, _H),
                       batch_p, Wlin, blin.reshape(1, _NPRED))
    return out


# speedup vs baseline: 1.0020x; 1.0020x over previous
import functools

import jax
import jax.numpy as jnp
from jax import lax
from jax.experimental import pallas as pl
from jax.experimental.pallas import tpu as pltpu
from jax.experimental.pallas import tpu_sc as plsc

_N = 10000
_NP = 10240
_E = 160000
_NG = 64
_NPRED = 10
_H = 512
_LANES = 128
_TILES = 16
_CORES = 2
_EB = 128
_NB = 80
_GB = 16
_NG_IDX = _NB // _GB
_EPT = _NB * _EB
_EPAD = _EPT * _TILES
_RPT = _NP // _TILES


def _make_segsum(C):
    mesh = plsc.VectorSubcoreMesh(core_axis_name="c", subcore_axis_name="s")

    @functools.partial(
        pl.kernel,
        mesh=mesh,
        out_type=jax.ShapeDtypeStruct((C, _NP, _LANES), jnp.float32),
        scratch_types=[
            pltpu.VMEM((2, _GB, _EB), jnp.int32),
            pltpu.VMEM((2, _GB, _EB), jnp.int32),
            pltpu.VMEM((2, _EB, _LANES), jnp.float32),
            pltpu.VMEM((32, _LANES), jnp.float32),
            pltpu.VMEM_SHARED((_NP, _LANES), jnp.float32),
            pltpu.SemaphoreType.DMA,
            pltpu.SemaphoreType.DMA,
            pltpu.SemaphoreType.DMA,
            pltpu.SemaphoreType.DMA,
        ],
    )
    def segsum(h_hbm, src_hbm, dst_hbm, out_hbm,
               src_v, dst_v, rows_v, zbuf, acc, sem0, sem1, semi0, semi1):
        sems = (sem0, sem1)
        semi = (semi0, semi1)
        cid = lax.axis_index("c")
        sid = lax.axis_index("s")

        def _stage(g, slot):
            pltpu.make_async_copy(src_hbm.at[sid].at[g],
                                  src_v.at[slot], semi[slot]).start()
            pltpu.make_async_copy(dst_hbm.at[sid].at[g],
                                  dst_v.at[slot], semi[slot]).start()

        def _stage_wait(g, slot):
            pltpu.make_async_copy(src_hbm.at[sid].at[g],
                                  src_v.at[slot], semi[slot]).wait()
            pltpu.make_async_copy(dst_hbm.at[sid].at[g],
                                  dst_v.at[slot], semi[slot]).wait()

        zv = jnp.zeros((16,), jnp.float32)

        def _zb(i, carry):
            r = i // (_LANES // 16)
            col = (i % (_LANES // 16)) * 16
            zbuf[r, pl.ds(col, 16)] = zv
            return carry

        lax.fori_loop(0, 32 * (_LANES // 16), _zb, 0)

        def _edge_loop(c):
            tbl = h_hbm.at[c]

            def _start(j, slot, buf):
                pltpu.make_async_copy(tbl.at[src_v.at[slot].at[j]],
                                      rows_v.at[buf], sems[buf]).start()

            def _finish(j, slot, buf):
                pltpu.make_async_copy(tbl.at[src_v.at[slot].at[j]],
                                      rows_v.at[buf], sems[buf]).wait()
                pltpu.sync_copy(rows_v.at[buf], acc.at[dst_v.at[slot].at[j]],
                                add=True)

            _stage(0, 0)
            for g in range(_NG_IDX):
                slot = g % 2
                _stage_wait(g, slot)
                if g + 1 < _NG_IDX:
                    _stage(g + 1, (g + 1) % 2)
                _start(0, slot, 0)

                def _pair(p, carry, slot=slot):
                    j0 = 2 * p
                    _start(j0 + 1, slot, 1)
                    _finish(j0, slot, 0)

                    @pl.when(j0 + 2 < _GB)
                    def _():
                        _start(j0 + 2, slot, 0)

                    _finish(j0 + 1, slot, 1)
                    return carry

                lax.fori_loop(0, _GB // 2, _pair, 0)

        def _writeout(c):
            base = sid * _RPT
            for j in range(_RPT // _EB):
                pltpu.sync_copy(acc.at[pl.ds(base + j * _EB, _EB)],
                                out_hbm.at[c].at[pl.ds(base + j * _EB, _EB)])

        for k in range(C // _CORES):
            base = sid * _RPT
            for j in range(_RPT // 32):
                pltpu.sync_copy(zbuf, acc.at[pl.ds(base + j * 32, 32)])
            plsc.subcore_barrier()

            for c in (2 * k, 2 * k + 1):
                pl.when(cid == (c % 2))(functools.partial(_edge_loop, c))
            plsc.subcore_barrier()

            for c in (2 * k, 2 * k + 1):
                pl.when(cid == (c % 2))(functools.partial(_writeout, c))
            if k + 1 < C // _CORES:
                plsc.subcore_barrier()

    return segsum


def _make_dense(C_in, relu, BM=1024):
    C_out = _H // _LANES

    def body(agg_ref, h_ref, wrel_ref, wroot_ref, b_ref, out_ref):
        s = jnp.zeros((BM, _H), jnp.float32)
        for c in range(C_in):
            s = s + jnp.dot(agg_ref[c].astype(jnp.bfloat16), wrel_ref[c],
                            preferred_element_type=jnp.float32)
            s = s + jnp.dot(h_ref[c].astype(jnp.bfloat16), wroot_ref[c],
                            preferred_element_type=jnp.float32)
        s = s + b_ref[...]
        if relu:
            s = jnp.maximum(s, 0.0)
        for co in range(C_out):
            out_ref[co] = s[:, co * _LANES:(co + 1) * _LANES]

    return pl.pallas_call(
        body,
        grid=(_NP // BM,),
        in_specs=[
            pl.BlockSpec((C_in, BM, _LANES), lambda i: (0, i, 0)),
            pl.BlockSpec((C_in, BM, _LANES), lambda i: (0, i, 0)),
            pl.BlockSpec((C_in, _LANES, _H), lambda i: (0, 0, 0)),
            pl.BlockSpec((C_in, _LANES, _H), lambda i: (0, 0, 0)),
            pl.BlockSpec((1, _H), lambda i: (0, 0)),
        ],
        out_specs=pl.BlockSpec((C_out, BM, _LANES), lambda i: (0, i, 0)),
        out_shape=jax.ShapeDtypeStruct((C_out, _NP, _LANES), jnp.float32),
    )


def _make_chunk2(BM=1024):
    def body(x_ref, out_ref):
        out_ref[0] = x_ref[...]

    return pl.pallas_call(
        body,
        grid=(2, _NP // BM),
        in_specs=[pl.BlockSpec((BM, _LANES), lambda c, i: (i, c))],
        out_specs=pl.BlockSpec((1, BM, _LANES), lambda c, i: (c, i, 0)),
        out_shape=jax.ShapeDtypeStruct((2, _NP, _LANES), jnp.float32),
    )


def _make_pool(BM=1024):
    nsteps = _NP // BM
    C_in = _H // _LANES

    def body(agg_ref, h_ref, wrel_ref, wroot_ref, brel_ref, batch_ref,
             wlin_ref, blin_ref, out_ref, sums_ref):
        i = pl.program_id(0)

        @pl.when(i == 0)
        def _():
            sums_ref[...] = jnp.zeros_like(sums_ref)

        s = jnp.zeros((BM, _H), jnp.float32)
        for c in range(C_in):
            s = s + jnp.dot(agg_ref[c].astype(jnp.bfloat16), wrel_ref[c],
                            preferred_element_type=jnp.float32)
            s = s + jnp.dot(h_ref[c].astype(jnp.bfloat16), wroot_ref[c],
                            preferred_element_type=jnp.float32)
        s = s + brel_ref[...]

        oh = (batch_ref[...] ==
              lax.broadcasted_iota(jnp.int32, (1, _NG), 1)).astype(jnp.float32)
        hcat = jnp.concatenate([s, jnp.ones((BM, _LANES), jnp.float32)],
                               axis=1)
        sums_ref[...] += lax.dot_general(
            oh, hcat, (((0,), (0,)), ((), ())),
            preferred_element_type=jnp.float32)

        @pl.when(i == nsteps - 1)
        def _():
            t = sums_ref[...]
            pooled = t[:, :_H] / jnp.maximum(t[:, _H:_H + 1], 1.0)
            out_ref[...] = (jnp.dot(pooled, wlin_ref[...],
                                    preferred_element_type=jnp.float32)
                            + blin_ref[...])

    return pl.pallas_call(
        body,
        grid=(nsteps,),
        in_specs=[
            pl.BlockSpec((C_in, BM, _LANES), lambda i: (0, i, 0)),
            pl.BlockSpec((C_in, BM, _LANES), lambda i: (0, i, 0)),
            pl.BlockSpec((C_in, _LANES, _H), lambda i: (0, 0, 0)),
            pl.BlockSpec((C_in, _LANES, _H), lambda i: (0, 0, 0)),
            pl.BlockSpec((1, _H), lambda i: (0, 0)),
            pl.BlockSpec((BM, 1), lambda i: (i, 0)),
            pl.BlockSpec((_H, _NPRED), lambda i: (0, 0)),
            pl.BlockSpec((1, _NPRED), lambda i: (0, 0)),
        ],
        out_specs=pl.BlockSpec((_NG, _NPRED), lambda i: (0, 0)),
        out_shape=jax.ShapeDtypeStruct((_NG, _NPRED), jnp.float32),
        scratch_shapes=[pltpu.VMEM((_NG, _H + _LANES), jnp.float32)],
    )


def kernel(x, edge_index, batch,
           Wrel1, brel1, Wroot1,
           Wrel2, brel2, Wroot2,
           Wrel3, brel3, Wroot3,
           Wlin, blin):
    src = edge_index[0].astype(jnp.int32)
    dst = edge_index[1].astype(jnp.int32)

    pad = _EPAD - _E
    ar = jnp.arange(pad, dtype=jnp.int32)
    src16 = jnp.concatenate([src, (ar * 997) % _N]).reshape(
        _TILES, _NG_IDX, _GB, _EB)
    dst16 = jnp.concatenate([dst, _N + (ar % 16)]).reshape(
        _TILES, _NG_IDX, _GB, _EB)

    x_pad = jnp.concatenate(
        [x, jnp.zeros((_NP - _N, x.shape[1]), jnp.float32)])
    x_c = _make_chunk2()(x_pad)
    batch_p = jnp.concatenate(
        [batch.astype(jnp.int32), jnp.full((_NP - _N,), _NG, jnp.int32)]
    ).reshape(_NP, 1)

    segsum2 = _make_segsum(2)
    segsum4 = _make_segsum(4)

    agg1 = segsum2(x_c, src16, dst16)
    h1 = _make_dense(2, True)(agg1, x_c,
                              Wrel1.astype(jnp.bfloat16).reshape(2, _LANES, _H),
                              Wroot1.astype(jnp.bfloat16).reshape(2, _LANES, _H),
                              brel1.reshape(1, _H))
    agg2 = segsum4(h1, src16, dst16)
    h2 = _make_dense(4, True)(agg2, h1,
                              Wrel2.astype(jnp.bfloat16).reshape(4, _LANES, _H),
                              Wroot2.astype(jnp.bfloat16).reshape(4, _LANES, _H),
                              brel2.reshape(1, _H))
    agg3 = segsum4(h2, src16, dst16)
    out = _make_pool()(agg3, h2,
                       Wrel3.astype(jnp.bfloat16).reshape(4, _LANES, _H),
                       Wroot3.astype(jnp.bfloat16).reshape(4, _LANES, _H),
                       brel3.reshape(1, _H),
                       batch_p, Wlin, blin.reshape(1, _NPRED))
    return out


# speedup vs baseline: 1.0069x; 1.0049x over previous
import functools

import jax
import jax.numpy as jnp
from jax import lax
from jax.experimental import pallas as pl
from jax.experimental.pallas import tpu as pltpu
from jax.experimental.pallas import tpu_sc as plsc

_N = 10000
_NP = 10240
_E = 160000
_NG = 64
_NPRED = 10
_H = 512
_LANES = 128
_TILES = 16
_CORES = 2
_EB = 128
_NB = 80
_GB = 16
_NG_IDX = _NB // _GB
_EPT = _NB * _EB
_EPAD = _EPT * _TILES
_RPT = _NP // _TILES


def _make_segsum(C):
    mesh = plsc.VectorSubcoreMesh(core_axis_name="c", subcore_axis_name="s")

    @functools.partial(
        pl.kernel,
        mesh=mesh,
        out_type=jax.ShapeDtypeStruct((C, _NP, _LANES), jnp.float32),
        scratch_types=[
            pltpu.VMEM((2, _GB, _EB), jnp.int32),
            pltpu.VMEM((2, _GB, _EB), jnp.int32),
            pltpu.VMEM((2, _EB, _LANES), jnp.float32),
            pltpu.VMEM((32, _LANES), jnp.float32),
            pltpu.VMEM_SHARED((_NP, _LANES), jnp.float32),
            pltpu.SemaphoreType.DMA,
            pltpu.SemaphoreType.DMA,
            pltpu.SemaphoreType.DMA,
            pltpu.SemaphoreType.DMA,
        ],
    )
    def segsum(h_hbm, src_hbm, dst_hbm, out_hbm,
               src_v, dst_v, rows_v, zbuf, acc, sem0, sem1, semi0, semi1):
        sems = (sem0, sem1)
        semi = (semi0, semi1)
        cid = lax.axis_index("c")
        sid = lax.axis_index("s")

        def _stage(g, slot):
            pltpu.make_async_copy(src_hbm.at[sid].at[g],
                                  src_v.at[slot], semi[slot]).start()
            pltpu.make_async_copy(dst_hbm.at[sid].at[g],
                                  dst_v.at[slot], semi[slot]).start()

        def _stage_wait(g, slot):
            pltpu.make_async_copy(src_hbm.at[sid].at[g],
                                  src_v.at[slot], semi[slot]).wait()
            pltpu.make_async_copy(dst_hbm.at[sid].at[g],
                                  dst_v.at[slot], semi[slot]).wait()

        zv = jnp.zeros((16,), jnp.float32)

        def _zb(i, carry):
            r = i // (_LANES // 16)
            col = (i % (_LANES // 16)) * 16
            zbuf[r, pl.ds(col, 16)] = zv
            return carry

        lax.fori_loop(0, 32 * (_LANES // 16), _zb, 0)

        def _edge_loop(c):
            tbl = h_hbm.at[c]

            def _start(j, slot, buf):
                pltpu.make_async_copy(tbl.at[src_v.at[slot].at[j]],
                                      rows_v.at[buf], sems[buf]).start()

            def _finish(j, slot, buf):
                pltpu.make_async_copy(tbl.at[src_v.at[slot].at[j]],
                                      rows_v.at[buf], sems[buf]).wait()
                pltpu.sync_copy(rows_v.at[buf], acc.at[dst_v.at[slot].at[j]],
                                add=True)

            _stage(0, 0)
            for g in range(_NG_IDX):
                slot = g % 2
                _stage_wait(g, slot)
                if g + 1 < _NG_IDX:
                    _stage(g + 1, (g + 1) % 2)
                _start(0, slot, 0)

                def _pair(p, carry, slot=slot):
                    j0 = 2 * p
                    _start(j0 + 1, slot, 1)
                    _finish(j0, slot, 0)

                    @pl.when(j0 + 2 < _GB)
                    def _():
                        _start(j0 + 2, slot, 0)

                    _finish(j0 + 1, slot, 1)
                    return carry

                lax.fori_loop(0, _GB // 2, _pair, 0)

        def _writeout(c):
            base = sid * _RPT
            for j in range(_RPT // _EB):
                pltpu.make_async_copy(
                    acc.at[pl.ds(base + j * _EB, _EB)],
                    out_hbm.at[c].at[pl.ds(base + j * _EB, _EB)],
                    sem0).start()
            for j in range(_RPT // _EB):
                pltpu.make_async_copy(
                    acc.at[pl.ds(base + j * _EB, _EB)],
                    out_hbm.at[c].at[pl.ds(base + j * _EB, _EB)],
                    sem0).wait()

        for k in range(C // _CORES):
            base = sid * _RPT
            for j in range(_RPT // 32):
                pltpu.make_async_copy(
                    zbuf, acc.at[pl.ds(base + j * 32, 32)], sem0).start()
            for j in range(_RPT // 32):
                pltpu.make_async_copy(
                    zbuf, acc.at[pl.ds(base + j * 32, 32)], sem0).wait()
            plsc.subcore_barrier()

            for c in (2 * k, 2 * k + 1):
                pl.when(cid == (c % 2))(functools.partial(_edge_loop, c))
            plsc.subcore_barrier()

            for c in (2 * k, 2 * k + 1):
                pl.when(cid == (c % 2))(functools.partial(_writeout, c))
            if k + 1 < C // _CORES:
                plsc.subcore_barrier()

    return segsum


def _make_dense(C_in, relu, BM=1024):
    C_out = _H // _LANES

    def body(agg_ref, h_ref, wrel_ref, wroot_ref, b_ref, out_ref):
        s = jnp.zeros((BM, _H), jnp.float32)
        for c in range(C_in):
            s = s + jnp.dot(agg_ref[c], wrel_ref[c],
                            preferred_element_type=jnp.float32)
            s = s + jnp.dot(h_ref[c], wroot_ref[c],
                            preferred_element_type=jnp.float32)
        s = s + b_ref[...]
        if relu:
            s = jnp.maximum(s, 0.0)
        for co in range(C_out):
            out_ref[co] = s[:, co * _LANES:(co + 1) * _LANES]

    return pl.pallas_call(
        body,
        grid=(_NP // BM,),
        in_specs=[
            pl.BlockSpec((C_in, BM, _LANES), lambda i: (0, i, 0)),
            pl.BlockSpec((C_in, BM, _LANES), lambda i: (0, i, 0)),
            pl.BlockSpec((C_in, _LANES, _H), lambda i: (0, 0, 0)),
            pl.BlockSpec((C_in, _LANES, _H), lambda i: (0, 0, 0)),
            pl.BlockSpec((1, _H), lambda i: (0, 0)),
        ],
        out_specs=pl.BlockSpec((C_out, BM, _LANES), lambda i: (0, i, 0)),
        out_shape=jax.ShapeDtypeStruct((C_out, _NP, _LANES), jnp.float32),
    )


def _make_chunk2(BM=1024):
    def body(x_ref, out_ref):
        out_ref[0] = x_ref[...]

    return pl.pallas_call(
        body,
        grid=(2, _NP // BM),
        in_specs=[pl.BlockSpec((BM, _LANES), lambda c, i: (i, c))],
        out_specs=pl.BlockSpec((1, BM, _LANES), lambda c, i: (c, i, 0)),
        out_shape=jax.ShapeDtypeStruct((2, _NP, _LANES), jnp.float32),
    )


def _make_pool(BM=1024):
    nsteps = _NP // BM
    C_in = _H // _LANES

    def body(agg_ref, h_ref, wrel_ref, wroot_ref, brel_ref, batch_ref,
             wlin_ref, blin_ref, out_ref, sums_ref):
        i = pl.program_id(0)

        @pl.when(i == 0)
        def _():
            sums_ref[...] = jnp.zeros_like(sums_ref)

        s = jnp.zeros((BM, _H), jnp.float32)
        for c in range(C_in):
            s = s + jnp.dot(agg_ref[c], wrel_ref[c],
                            preferred_element_type=jnp.float32)
            s = s + jnp.dot(h_ref[c], wroot_ref[c],
                            preferred_element_type=jnp.float32)
        s = s + brel_ref[...]

        oh = (batch_ref[...] ==
              lax.broadcasted_iota(jnp.int32, (1, _NG), 1)).astype(jnp.float32)
        hcat = jnp.concatenate([s, jnp.ones((BM, _LANES), jnp.float32)],
                               axis=1)
        sums_ref[...] += lax.dot_general(
            oh, hcat, (((0,), (0,)), ((), ())),
            preferred_element_type=jnp.float32)

        @pl.when(i == nsteps - 1)
        def _():
            t = sums_ref[...]
            pooled = t[:, :_H] / jnp.maximum(t[:, _H:_H + 1], 1.0)
            out_ref[...] = (jnp.dot(pooled, wlin_ref[...],
                                    preferred_element_type=jnp.float32)
                            + blin_ref[...])

    return pl.pallas_call(
        body,
        grid=(nsteps,),
        in_specs=[
            pl.BlockSpec((C_in, BM, _LANES), lambda i: (0, i, 0)),
            pl.BlockSpec((C_in, BM, _LANES), lambda i: (0, i, 0)),
            pl.BlockSpec((C_in, _LANES, _H), lambda i: (0, 0, 0)),
            pl.BlockSpec((C_in, _LANES, _H), lambda i: (0, 0, 0)),
            pl.BlockSpec((1, _H), lambda i: (0, 0)),
            pl.BlockSpec((BM, 1), lambda i: (i, 0)),
            pl.BlockSpec((_H, _NPRED), lambda i: (0, 0)),
            pl.BlockSpec((1, _NPRED), lambda i: (0, 0)),
        ],
        out_specs=pl.BlockSpec((_NG, _NPRED), lambda i: (0, 0)),
        out_shape=jax.ShapeDtypeStruct((_NG, _NPRED), jnp.float32),
        scratch_shapes=[pltpu.VMEM((_NG, _H + _LANES), jnp.float32)],
    )


def kernel(x, edge_index, batch,
           Wrel1, brel1, Wroot1,
           Wrel2, brel2, Wroot2,
           Wrel3, brel3, Wroot3,
           Wlin, blin):
    src = edge_index[0].astype(jnp.int32)
    dst = edge_index[1].astype(jnp.int32)

    pad = _EPAD - _E
    ar = jnp.arange(pad, dtype=jnp.int32)
    src16 = jnp.concatenate([src, (ar * 997) % _N]).reshape(
        _TILES, _NG_IDX, _GB, _EB)
    dst16 = jnp.concatenate([dst, _N + (ar % 16)]).reshape(
        _TILES, _NG_IDX, _GB, _EB)

    x_pad = jnp.concatenate(
        [x, jnp.zeros((_NP - _N, x.shape[1]), jnp.float32)])
    x_c = _make_chunk2()(x_pad)
    batch_p = jnp.concatenate(
        [batch.astype(jnp.int32), jnp.full((_NP - _N,), _NG, jnp.int32)]
    ).reshape(_NP, 1)

    segsum2 = _make_segsum(2)
    segsum4 = _make_segsum(4)

    agg1 = segsum2(x_c, src16, dst16)
    h1 = _make_dense(2, True)(agg1, x_c,
                              Wrel1.reshape(2, _LANES, _H),
                              Wroot1.reshape(2, _LANES, _H),
                              brel1.reshape(1, _H))
    agg2 = segsum4(h1, src16, dst16)
    h2 = _make_dense(4, True)(agg2, h1,
                              Wrel2.reshape(4, _LANES, _H),
                              Wroot2.reshape(4, _LANES, _H),
                              brel2.reshape(1, _H))
    agg3 = segsum4(h2, src16, dst16)
    out = _make_pool()(agg3, h2,
                       Wrel3.reshape(4, _LANES, _H),
                       Wroot3.reshape(4, _LANES, _H),
                       brel3.reshape(1, _H),
                       batch_p, Wlin, blin.reshape(1, _NPRED))
    return out
